# baseline (speedup 1.0000x reference)
import functools

import jax
import jax.numpy as jnp
from jax import lax
from jax.experimental import pallas as pl
from jax.experimental.pallas import tpu as pltpu
from jax.experimental.pallas import tpu_sc as plsc

B = 4
S = 8192
N_EMBD = 768
N_CLASS = 10
CLF_TOKEN = 40480
LANES = 16
SCAN_PER_SEQ = 3
CHUNK_A = 2816
CHUNK_B = 2560

_GATHER_DNUMS = lax.GatherDimensionNumbers(
    offset_dims=(), collapsed_slice_dims=(0,), start_index_map=(0,))


def _lane_shuffle(x, idx):
    return lax.gather(x, idx[:, None], _GATHER_DNUMS, (1,),
                      mode=lax.GatherScatterMode.PROMISE_IN_BOUNDS)


def _lane_allreduce_sum(x, lane_iota):
    for shift in (8, 4, 2, 1):
        idx = (lane_iota + shift) & (LANES - 1)
        x = x + _lane_shuffle(x, idx)
    return x


def _dense_head(row_v, wt_v, bias_v, lane_iota, unroll):
    def mm_step(i, accs):
        rv = row_v[pl.ds(i * LANES, LANES)]
        return tuple(
            accs[j] + rv * wt_v[j, pl.ds(i * LANES, LANES)]
            for j in range(N_CLASS))

    accs = lax.fori_loop(
        0, N_EMBD // LANES, mm_step,
        tuple(jnp.zeros((LANES,), jnp.float32) for _ in range(N_CLASS)),
        unroll=unroll)
    logits = bias_v[...]
    for j in range(N_CLASS):
        colsum = _lane_allreduce_sum(accs[j], lane_iota)
        logits = jnp.where(lane_iota == j, logits + colsum, logits)
    return logits


def _clf_body(h2_hbm, tok_hbm, wt_hbm, bias_hbm, out_hbm,
              tok_v, row_v, wt_v, bias_v, out_v, cnt_smem,
              wsem, bsem, tsem, rsem):
    s = lax.axis_index("s")
    lane_iota = lax.iota(jnp.int32, LANES)
    is_owner = s < B
    k = s - B
    sb = k // SCAN_PER_SEQ
    t = k % SCAN_PER_SEQ
    qbase = t * CHUNK_A

    cnt_smem[0] = 0

    @pl.when(jnp.logical_and(jnp.logical_not(is_owner), t < 2))
    def _():
        pltpu.async_copy(tok_hbm.at[sb, pl.ds(qbase, CHUNK_A)], tok_v, tsem)

    @pl.when(jnp.logical_and(jnp.logical_not(is_owner), t == 2))
    def _():
        pltpu.async_copy(tok_hbm.at[sb, pl.ds(qbase, CHUNK_B)],
                         tok_v.at[pl.ds(0, CHUNK_B)], tsem)

    @pl.when(is_owner)
    def _():
        pltpu.async_copy(wt_hbm, wt_v, wsem)
        pltpu.async_copy(bias_hbm, bias_v, bsem)
        pltpu.async_copy(h2_hbm.at[s * S + (S - 1)], row_v, rsem)

    plsc.subcore_barrier()

    @pl.when(jnp.logical_not(is_owner))
    def _():
        @pl.when(t < 2)
        def _():
            pltpu.make_async_copy(tok_hbm.at[sb, pl.ds(qbase, CHUNK_A)],
                                  tok_v, tsem).wait()

        @pl.when(t == 2)
        def _():
            pltpu.make_async_copy(tok_hbm.at[sb, pl.ds(qbase, CHUNK_B)],
                                  tok_v.at[pl.ds(0, CHUNK_B)], tsem).wait()
            zv = jnp.zeros((LANES,), jnp.int32)
            for z in range((CHUNK_A - CHUNK_B) // LANES):
                tok_v[pl.ds(CHUNK_B + z * LANES, LANES)] = zv

        def scan_step(i, vacc):
            tv = tok_v[pl.ds(i * LANES, LANES)]
            posv = lane_iota + (qbase + i * LANES)
            return vacc + jnp.where(tv == CLF_TOKEN, posv, 0)

        vacc = lax.fori_loop(0, CHUNK_A // LANES, scan_step,
                             jnp.zeros((LANES,), jnp.int32), unroll=8)
        part = _lane_allreduce_sum(vacc, lane_iota)
        p = jnp.squeeze(lax.slice(part, (0,), (1,)))
        plsc.fetch_and_add(cnt_smem.at[0], p, subcore_id=sb)

    @pl.when(is_owner)
    def _():
        pltpu.make_async_copy(h2_hbm.at[s * S + (S - 1)], row_v, rsem).wait()
        pltpu.make_async_copy(wt_hbm, wt_v, wsem).wait()
        pltpu.make_async_copy(bias_hbm, bias_v, bsem).wait()
        out_v[...] = _dense_head(row_v, wt_v, bias_v, lane_iota, 8)

    plsc.subcore_barrier()

    @pl.when(is_owner)
    def _():
        idx = cnt_smem[0] + s * S
        idx = jnp.minimum(jnp.maximum(idx, 0), B * S - 1)

        @pl.when(idx != s * S + (S - 1))
        def _():
            pltpu.sync_copy(h2_hbm.at[idx], row_v)
            out_v[...] = _dense_head(row_v, wt_v, bias_v, lane_iota, 2)

        pltpu.sync_copy(out_v, out_hbm.at[s])


@jax.jit
def kernel(h, x, W, b):
    h2 = h.reshape(B * S, N_EMBD)
    tok = x[..., 0]
    wt = W.T
    bias_pad = jnp.zeros((LANES,), jnp.float32).at[:N_CLASS].set(b)

    mesh = plsc.VectorSubcoreMesh(core_axis_name="c", subcore_axis_name="s",
                                  num_cores=1)
    run = functools.partial(
        pl.kernel,
        mesh=mesh,
        out_type=jax.ShapeDtypeStruct((B, LANES), jnp.float32),
        scratch_types=[
            pltpu.VMEM((CHUNK_A,), jnp.int32),
            pltpu.VMEM((N_EMBD,), jnp.float32),
            pltpu.VMEM((N_CLASS, N_EMBD), jnp.float32),
            pltpu.VMEM((LANES,), jnp.float32),
            pltpu.VMEM((LANES,), jnp.float32),
            pltpu.SMEM((1,), jnp.int32),
            pltpu.SemaphoreType.DMA,
            pltpu.SemaphoreType.DMA,
            pltpu.SemaphoreType.DMA,
            pltpu.SemaphoreType.DMA,
        ],
    )(_clf_body)
    out = run(h2, tok, wt, bias_pad)
    return out[:, :N_CLASS]


# speedup vs baseline: 1.0338x; 1.0338x over previous
import functools

import jax
import jax.numpy as jnp
from jax import lax
from jax.experimental import pallas as pl
from jax.experimental.pallas import tpu as pltpu
from jax.experimental.pallas import tpu_sc as plsc

B = 4
S = 8192
N_EMBD = 768
N_CLASS = 10
CLF_TOKEN = 40480
LANES = 16
QUARTERS = 4
QCHUNK = S // QUARTERS

_GATHER_DNUMS = lax.GatherDimensionNumbers(
    offset_dims=(), collapsed_slice_dims=(0,), start_index_map=(0,))


def _lane_shuffle(x, idx):
    return lax.gather(x, idx[:, None], _GATHER_DNUMS, (1,),
                      mode=lax.GatherScatterMode.PROMISE_IN_BOUNDS)


def _lane_allreduce_sum(x, lane_iota):
    for shift in (8, 4, 2, 1):
        idx = (lane_iota + shift) & (LANES - 1)
        x = x + _lane_shuffle(x, idx)
    return x


def _clf_body(h2_hbm, tok_hbm, wt_hbm, bias_hbm, out_hbm,
              tok_v, row_v, wt_v, bias_v, out_v, cnt_smem,
              wsem, bsem, tsem, rsem):
    s = lax.axis_index("s")
    lane_iota = lax.iota(jnp.int32, LANES)
    b = s // QUARTERS
    q = s % QUARTERS
    owner = b * QUARTERS

    cnt_smem[0] = 0
    pltpu.async_copy(tok_hbm.at[b, pl.ds(q * QCHUNK, QCHUNK)], tok_v, tsem)

    @pl.when(q == 0)
    def _():
        pltpu.async_copy(wt_hbm, wt_v, wsem)
        pltpu.async_copy(bias_hbm, bias_v, bsem)
        pltpu.async_copy(h2_hbm.at[b * S + (S - 1)], row_v, rsem)

    plsc.subcore_barrier()

    pltpu.make_async_copy(tok_hbm.at[b, pl.ds(q * QCHUNK, QCHUNK)],
                          tok_v, tsem).wait()
    qbase = q * QCHUNK

    def scan_step(i, vacc):
        tv = tok_v[pl.ds(i * LANES, LANES)]
        posv = lane_iota + (qbase + i * LANES)
        return vacc + jnp.where(tv == CLF_TOKEN, posv, 0)

    vacc = lax.fori_loop(0, QCHUNK // LANES, scan_step,
                         jnp.zeros((LANES,), jnp.int32), unroll=8)
    part = _lane_allreduce_sum(vacc, lane_iota)
    p = jnp.squeeze(lax.slice(part, (0,), (1,)))
    plsc.fetch_and_add(cnt_smem.at[0], p, subcore_id=owner)

    plsc.subcore_barrier()

    @pl.when(q == 0)
    def _():
        idx = cnt_smem[0] + b * S
        idx = jnp.minimum(jnp.maximum(idx, 0), B * S - 1)
        pltpu.make_async_copy(h2_hbm.at[b * S + (S - 1)], row_v, rsem).wait()

        @pl.when(idx != b * S + (S - 1))
        def _():
            pltpu.sync_copy(h2_hbm.at[idx], row_v)

        pltpu.make_async_copy(wt_hbm, wt_v, wsem).wait()
        pltpu.make_async_copy(bias_hbm, bias_v, bsem).wait()

        def mm_step(i, accs):
            rv = row_v[pl.ds(i * LANES, LANES)]
            return tuple(
                accs[j] + rv * wt_v[j, pl.ds(i * LANES, LANES)]
                for j in range(N_CLASS))

        accs = lax.fori_loop(
            0, N_EMBD // LANES, mm_step,
            tuple(jnp.zeros((LANES,), jnp.float32) for _ in range(N_CLASS)),
            unroll=8)

        logits = bias_v[...]
        for j in range(N_CLASS):
            colsum = _lane_allreduce_sum(accs[j], lane_iota)
            logits = jnp.where(lane_iota == j, logits + colsum, logits)
        out_v[...] = logits
        pltpu.sync_copy(out_v, out_hbm.at[b])


@jax.jit
def kernel(h, x, W, b):
    h2 = h.reshape(B * S, N_EMBD)
    tok = x[..., 0]
    wt = W.T
    bias_pad = jnp.zeros((LANES,), jnp.float32).at[:N_CLASS].set(b)

    mesh = plsc.VectorSubcoreMesh(core_axis_name="c", subcore_axis_name="s",
                                  num_cores=1)
    run = functools.partial(
        pl.kernel,
        mesh=mesh,
        out_type=jax.ShapeDtypeStruct((B, LANES), jnp.float32),
        scratch_types=[
            pltpu.VMEM((QCHUNK,), jnp.int32),
            pltpu.VMEM((N_EMBD,), jnp.float32),
            pltpu.VMEM((N_CLASS, N_EMBD), jnp.float32),
            pltpu.VMEM((LANES,), jnp.float32),
            pltpu.VMEM((LANES,), jnp.float32),
            pltpu.SMEM((1,), jnp.int32),
            pltpu.SemaphoreType.DMA,
            pltpu.SemaphoreType.DMA,
            pltpu.SemaphoreType.DMA,
            pltpu.SemaphoreType.DMA,
        ],
    )(_clf_body)
    out = run(h2, tok, wt, bias_pad)
    return out[:, :N_CLASS]
